# Initial kernel scaffold; baseline (speedup 1.0000x reference)
import jax
import jax.numpy as jnp
from jax.experimental import pallas as pl


def kernel(x, w1, wh, ww):
    raise NotImplementedError("write your pallas kernel here")


# speedup vs baseline: 1.1167x; 1.1167x over previous
import jax
import jax.numpy as jnp
from jax.experimental import pallas as pl
from jax.experimental.pallas import tpu as pltpu


def _ca_kernel(x_ref, p_ref, eh_ref, ew_ref, w1_ref, wh_ref, ww_ref, o_ref,
               *, H, W):
    f32 = jnp.float32
    bf16 = jnp.bfloat16
    B, C, HW = x_ref.shape

    xf = x_ref[...].reshape(B * C, HW)
    xbf = xf.astype(bf16)

    pooled = jnp.dot(xbf, p_ref[...], preferred_element_type=f32)

    sh_parts = []
    sw_parts = []
    for b in range(B):
        pb = pooled[b * C:(b + 1) * C]
        z = jnp.maximum(
            jnp.dot(w1_ref[...], pb, preferred_element_type=f32), 0.0)
        th = jnp.dot(wh_ref[...], z[:, :H], preferred_element_type=f32)
        tw = jnp.dot(ww_ref[...], z[:, H:], preferred_element_type=f32)
        sh_parts.append(jax.nn.sigmoid(th).astype(bf16))
        sw_parts.append(jax.nn.sigmoid(tw).astype(bf16))
    sh = jnp.concatenate(sh_parts, axis=0)
    sw = jnp.concatenate(sw_parts, axis=0)

    gh = jnp.dot(sh, eh_ref[...], preferred_element_type=f32)
    gw = jnp.dot(sw, ew_ref[...], preferred_element_type=f32)
    o_ref[...] = (xf * (gh * gw)).reshape(B, C, HW).astype(o_ref.dtype)


def kernel(x, w1, wh, ww):
    N, C, H, W = x.shape
    HW = H * W
    Cr = w1.shape[0]
    f32 = jnp.float32
    bf16 = jnp.bfloat16

    B = 4
    while N % B:
        B //= 2

    x_flat = x.reshape(N, C, HW)

    hw = jnp.arange(HW)
    oh_h = (hw[:, None] // W == jnp.arange(H)[None, :]).astype(f32)
    oh_w = (hw[:, None] % W == jnp.arange(W)[None, :]).astype(f32)
    p = jnp.concatenate([oh_h * (1.0 / W), oh_w * (1.0 / H)],
                        axis=1).astype(bf16)
    eh = oh_h.T.astype(bf16)
    ew = oh_w.T.astype(bf16)

    import functools
    body = functools.partial(_ca_kernel, H=H, W=W)

    flops = int(N * (2 * C * HW * (H + W)
                     + 2 * Cr * C * (H + W)
                     + 4 * C * Cr * (H + W)
                     + 2 * C * HW * (H + W)
                     + 2 * C * HW))
    bytes_acc = int(2 * N * C * HW * 4 + 3 * (H + W) * HW * 2 + 3 * C * Cr * 4)

    out_flat = pl.pallas_call(
        body,
        out_shape=jax.ShapeDtypeStruct((N, C, HW), x.dtype),
        grid=(N // B,),
        in_specs=[
            pl.BlockSpec((B, C, HW), lambda n: (n, 0, 0)),
            pl.BlockSpec((HW, H + W), lambda n: (0, 0)),
            pl.BlockSpec((H, HW), lambda n: (0, 0)),
            pl.BlockSpec((W, HW), lambda n: (0, 0)),
            pl.BlockSpec((Cr, C), lambda n: (0, 0)),
            pl.BlockSpec((C, Cr), lambda n: (0, 0)),
            pl.BlockSpec((C, Cr), lambda n: (0, 0)),
        ],
        out_specs=pl.BlockSpec((B, C, HW), lambda n: (n, 0, 0)),
        compiler_params=pltpu.CompilerParams(
            dimension_semantics=("parallel",),
            vmem_limit_bytes=56 << 20),
        cost_estimate=pl.CostEstimate(
            flops=flops,
            transcendentals=int(N * C * (H + W)),
            bytes_accessed=bytes_acc),
    )(x_flat, p, eh, ew, w1, wh, ww)

    return out_flat.reshape(N, C, H, W)


# speedup vs baseline: 3.6901x; 3.3045x over previous
import functools

import numpy as np
import jax
import jax.numpy as jnp
from jax import lax
from jax.experimental import pallas as pl
from jax.experimental.pallas import tpu as pltpu


def _ca_kernel(x_ref, pt_ref, w1_ref, wh_ref, ww_ref, o_ref, *, H, W):
    f32 = jnp.float32
    B = x_ref.shape[0]
    C = x_ref.shape[3]
    HW = H * W

    x4 = x_ref[...]
    x2 = x4.reshape(B * HW, C)

    for b in range(B):
        xb = x2[b * HW:(b + 1) * HW]
        pooled = jnp.dot(pt_ref[...], xb,
                         preferred_element_type=f32)
        z = jnp.maximum(
            lax.dot_general(pooled, w1_ref[...],
                            (((1,), (1,)), ((), ())),
                            preferred_element_type=f32), 0.0)
        sh = jax.nn.sigmoid(
            lax.dot_general(z[:H], wh_ref[...],
                            (((1,), (1,)), ((), ())),
                            preferred_element_type=f32))
        sw = jax.nn.sigmoid(
            lax.dot_general(z[H:], ww_ref[...],
                            (((1,), (1,)), ((), ())),
                            preferred_element_type=f32))
        o_ref[b] = x4[b] * sh[:, None, :] * sw[None, :, :]


def kernel(x, w1, wh, ww):
    N, C, H, W = x.shape
    HW = H * W
    Cr = w1.shape[0]

    B = 4
    while N % B:
        B //= 2

    xt = jnp.transpose(x, (0, 2, 3, 1))

    p = np.arange(HW)
    pt = np.concatenate(
        [(p[None, :] // W == np.arange(H)[:, None]) / W,
         (p[None, :] % W == np.arange(W)[:, None]) / H],
        axis=0).astype(np.float32)

    body = functools.partial(_ca_kernel, H=H, W=W)

    flops = int(N * (2 * HW * (H + W) * C
                     + 2 * (H + W) * C * Cr * 3
                     + 2 * HW * C))
    bytes_acc = int(2 * N * C * HW * 4 + (H + W) * HW * 4 + 3 * C * Cr * 4)

    out_t = pl.pallas_call(
        body,
        out_shape=jax.ShapeDtypeStruct((N, H, W, C), x.dtype),
        grid=(N // B,),
        in_specs=[
            pl.BlockSpec((B, H, W, C), lambda n: (n, 0, 0, 0)),
            pl.BlockSpec((H + W, HW), lambda n: (0, 0)),
            pl.BlockSpec((Cr, C), lambda n: (0, 0)),
            pl.BlockSpec((C, Cr), lambda n: (0, 0)),
            pl.BlockSpec((C, Cr), lambda n: (0, 0)),
        ],
        out_specs=pl.BlockSpec((B, H, W, C), lambda n: (n, 0, 0, 0)),
        compiler_params=pltpu.CompilerParams(
            dimension_semantics=("parallel",),
            vmem_limit_bytes=48 << 20),
        cost_estimate=pl.CostEstimate(
            flops=flops,
            transcendentals=int(N * C * (H + W)),
            bytes_accessed=bytes_acc),
    )(xt, jnp.asarray(pt), w1, wh, ww)

    return jnp.transpose(out_t, (0, 3, 1, 2))
